# Initial kernel scaffold; baseline (speedup 1.0000x reference)
import functools

import jax
import jax.numpy as jnp
from jax import lax
from jax.experimental import pallas as pl
from jax.experimental.pallas import tpu as pltpu
from jax.experimental.pallas import tpu_sc as plsc

_INFO = plsc.get_sparse_core_info()
_NC = _INFO.num_cores
_NS = _INFO.num_subcores
_NW = _NC * _NS

_IROW = 128
_G = 4


def _make_gather(n_idx_rows: int, vocab: int, dim: int):
  rows_per_worker = n_idx_rows // _NW
  n_groups = rows_per_worker // _G
  mesh = plsc.VectorSubcoreMesh(core_axis_name="c", subcore_axis_name="s")

  @functools.partial(
      pl.kernel,
      mesh=mesh,
      out_type=jax.ShapeDtypeStruct((n_idx_rows * _IROW, dim), jnp.float32),
      scratch_types=[
          pltpu.VMEM((rows_per_worker, _IROW), jnp.int32),
          pltpu.VMEM((_G * _IROW, dim), jnp.float32),
          pltpu.SemaphoreType.DMA,
      ],
  )
  def gather_kernel(idx_hbm, table_hbm, out_hbm, idx_v, rows_v, gsem):
    wid = lax.axis_index("s") * _NC + lax.axis_index("c")
    idx_base = wid * rows_per_worker
    out_base = idx_base * _IROW

    pltpu.sync_copy(idx_hbm.at[pl.ds(idx_base, rows_per_worker)], idx_v)

    @pl.loop(0, n_groups)
    def _group(g):
      copies = []
      for b in range(_G):
        j = g * _G + b
        copies.append(
            pltpu.async_copy(
                table_hbm.at[idx_v.at[j]],
                rows_v.at[pl.ds(b * _IROW, _IROW)],
                gsem,
            )
        )
      for c in copies:
        c.wait()
      pltpu.sync_copy(
          rows_v, out_hbm.at[pl.ds(out_base + g * (_G * _IROW), _G * _IROW)]
      )

  return gather_kernel


def kernel(input_ids, weight):
  bsz, seq = input_ids.shape
  vocab, dim = weight.shape
  n = bsz * seq
  assert n % (_NW * _G * _IROW) == 0
  idx2d = input_ids.reshape(n // _IROW, _IROW)
  out = _make_gather(n // _IROW, vocab, dim)(idx2d, weight)
  return out.reshape(bsz, seq, dim)


# speedup vs baseline: 4.0825x; 4.0825x over previous
import functools

import jax
import jax.numpy as jnp
from jax import lax
from jax.experimental import pallas as pl
from jax.experimental.pallas import tpu as pltpu
from jax.experimental.pallas import tpu_sc as plsc

_INFO = plsc.get_sparse_core_info()
_NC = _INFO.num_cores
_NS = _INFO.num_subcores
_NW = _NC * _NS

_IROW = 128
_G = 4


def _make_gather(n_idx_rows: int, vocab: int, dim: int):
  rows_per_worker = n_idx_rows // _NW
  n_groups = rows_per_worker // _G
  mesh = plsc.VectorSubcoreMesh(core_axis_name="c", subcore_axis_name="s")

  @functools.partial(
      pl.kernel,
      mesh=mesh,
      out_type=jax.ShapeDtypeStruct((n_idx_rows * _IROW, dim), jnp.float32),
      scratch_types=[
          pltpu.VMEM((rows_per_worker, _IROW), jnp.int32),
          pltpu.VMEM((_G * _IROW, dim), jnp.float32),
          pltpu.SemaphoreType.DMA,
      ],
      compiler_params=pltpu.CompilerParams(use_tc_tiling_on_sc=False),
  )
  def gather_kernel(idx_hbm, table_hbm, out_hbm, idx_v, rows_v, gsem):
    wid = lax.axis_index("s") * _NC + lax.axis_index("c")
    idx_base = wid * rows_per_worker
    out_base = idx_base * _IROW

    pltpu.sync_copy(idx_hbm.at[pl.ds(idx_base, rows_per_worker)], idx_v)

    @pl.loop(0, n_groups)
    def _group(g):
      copies = []
      for b in range(_G):
        j = g * _G + b
        copies.append(
            pltpu.async_copy(
                table_hbm.at[idx_v.at[j]],
                rows_v.at[pl.ds(b * _IROW, _IROW)],
                gsem,
            )
        )
      for c in copies:
        c.wait()
      pltpu.sync_copy(
          rows_v, out_hbm.at[pl.ds(out_base + g * (_G * _IROW), _G * _IROW)]
      )

  return gather_kernel


def kernel(input_ids, weight):
  bsz, seq = input_ids.shape
  vocab, dim = weight.shape
  n = bsz * seq
  assert n % (_NW * _G * _IROW) == 0
  idx2d = input_ids.reshape(n // _IROW, _IROW)
  out = _make_gather(n // _IROW, vocab, dim)(idx2d, weight)
  return out.reshape(bsz, seq, dim)


# speedup vs baseline: 4.2158x; 1.0327x over previous
import functools

import jax
import jax.numpy as jnp
from jax import lax
from jax.experimental import pallas as pl
from jax.experimental.pallas import tpu as pltpu
from jax.experimental.pallas import tpu_sc as plsc

_INFO = plsc.get_sparse_core_info()
_NC = _INFO.num_cores
_NS = _INFO.num_subcores
_NW = _NC * _NS

_IROW = 128
_G = 4


def _make_gather(n_idx_rows: int, vocab: int, dim: int):
  rows_per_worker = n_idx_rows // _NW
  n_groups = rows_per_worker // _G
  mesh = plsc.VectorSubcoreMesh(core_axis_name="c", subcore_axis_name="s")

  group_rows = _G * _IROW
  assert n_groups % 2 == 0 and n_groups >= 4

  @functools.partial(
      pl.kernel,
      mesh=mesh,
      out_type=jax.ShapeDtypeStruct((n_idx_rows * _IROW, dim), jnp.float32),
      scratch_types=[
          pltpu.VMEM((rows_per_worker, _IROW), jnp.int32),
          pltpu.VMEM((group_rows, dim), jnp.float32),
          pltpu.VMEM((group_rows, dim), jnp.float32),
          pltpu.SemaphoreType.DMA,
          pltpu.SemaphoreType.DMA,
          pltpu.SemaphoreType.DMA,
          pltpu.SemaphoreType.DMA,
      ],
      compiler_params=pltpu.CompilerParams(use_tc_tiling_on_sc=False),
  )
  def gather_kernel(idx_hbm, table_hbm, out_hbm,
                    idx_v, rows0, rows1, gsem0, gsem1, ssem0, ssem1):
    wid = lax.axis_index("s") * _NC + lax.axis_index("c")
    idx_base = wid * rows_per_worker
    out_base = idx_base * _IROW
    bufs = (rows0, rows1)
    gsems = (gsem0, gsem1)
    ssems = (ssem0, ssem1)

    pltpu.sync_copy(idx_hbm.at[pl.ds(idx_base, rows_per_worker)], idx_v)

    def fire_gather(g, p):
      for b in range(_G):
        pltpu.async_copy(
            table_hbm.at[idx_v.at[g * _G + b]],
            bufs[p].at[pl.ds(b * _IROW, _IROW)],
            gsems[p],
        )

    def drain_gather(g, p):
      for b in range(_G):
        pltpu.make_async_copy(
            table_hbm.at[idx_v.at[g * _G + b]],
            bufs[p].at[pl.ds(b * _IROW, _IROW)],
            gsems[p],
        ).wait()

    def fire_store(g, p):
      pltpu.async_copy(
          bufs[p], out_hbm.at[pl.ds(out_base + g * group_rows, group_rows)],
          ssems[p],
      )

    def wait_store(g, p):
      pltpu.make_async_copy(
          bufs[p], out_hbm.at[pl.ds(out_base + g * group_rows, group_rows)],
          ssems[p],
      ).wait()

    fire_gather(0, 0)
    fire_gather(1, 1)

    @pl.loop(0, n_groups - 2, step=2)
    def _pair(g):
      drain_gather(g, 0)
      fire_store(g, 0)
      drain_gather(g + 1, 1)
      fire_store(g + 1, 1)
      wait_store(g, 0)
      fire_gather(g + 2, 0)
      wait_store(g + 1, 1)
      fire_gather(g + 3, 1)

    g_last = n_groups - 2
    drain_gather(g_last, 0)
    fire_store(g_last, 0)
    drain_gather(g_last + 1, 1)
    fire_store(g_last + 1, 1)
    wait_store(g_last, 0)
    wait_store(g_last + 1, 1)

  return gather_kernel


def kernel(input_ids, weight):
  bsz, seq = input_ids.shape
  vocab, dim = weight.shape
  n = bsz * seq
  assert n % (_NW * _G * _IROW) == 0
  idx2d = input_ids.reshape(n // _IROW, _IROW)
  out = _make_gather(n // _IROW, vocab, dim)(idx2d, weight)
  return out.reshape(bsz, seq, dim)


# speedup vs baseline: 5.5561x; 1.3179x over previous
import functools

import jax
import jax.numpy as jnp
from jax import lax
from jax.experimental import pallas as pl
from jax.experimental.layout import Layout, with_layout_constraint
from jax.experimental.pallas import tpu as pltpu
from jax.experimental.pallas import tpu_sc as plsc

_INFO = plsc.get_sparse_core_info()
_NC = _INFO.num_cores
_NS = _INFO.num_subcores
_NW = _NC * _NS

_IROW = 128
_G = 4


def _make_gather(n_idx_rows: int, vocab: int, dim: int):
  rows_per_worker = n_idx_rows // _NW
  n_groups = rows_per_worker // _G
  mesh = plsc.VectorSubcoreMesh(core_axis_name="c", subcore_axis_name="s")

  group_rows = _G * _IROW
  group_rows128 = group_rows * dim // 128
  rows128_per_worker = rows_per_worker * _IROW * dim // 128
  assert n_groups % 2 == 0 and n_groups >= 4

  @functools.partial(
      pl.kernel,
      mesh=mesh,
      out_type=jax.ShapeDtypeStruct((n_idx_rows * _IROW, dim), jnp.float32),
      scratch_types=[
          pltpu.VMEM((rows_per_worker, _IROW), jnp.int32),
          pltpu.VMEM((group_rows, dim), jnp.float32),
          pltpu.VMEM((group_rows, dim), jnp.float32),
          pltpu.SemaphoreType.DMA,
          pltpu.SemaphoreType.DMA,
          pltpu.SemaphoreType.DMA,
          pltpu.SemaphoreType.DMA,
      ],
      compiler_params=pltpu.CompilerParams(use_tc_tiling_on_sc=False),
  )
  def gather_kernel(idx_hbm, table_hbm, out_hbm,
                    idx_v, rows0, rows1, gsem0, gsem1, ssem0, ssem1):
    wid = lax.axis_index("s") * _NC + lax.axis_index("c")
    idx_base = wid * rows_per_worker
    out_base = idx_base * _IROW
    bufs = (rows0, rows1)
    gsems = (gsem0, gsem1)
    ssems = (ssem0, ssem1)

    pltpu.sync_copy(idx_hbm.at[pl.ds(idx_base, rows_per_worker)], idx_v)

    def fire_gather(g, p):
      for b in range(_G):
        pltpu.async_copy(
            table_hbm.at[idx_v.at[g * _G + b]],
            bufs[p].at[pl.ds(b * _IROW, _IROW)],
            gsems[p],
        )

    def drain_gather(g, p):
      for b in range(_G):
        pltpu.make_async_copy(
            table_hbm.at[idx_v.at[g * _G + b]],
            bufs[p].at[pl.ds(b * _IROW, _IROW)],
            gsems[p],
        ).wait()

    def fire_store(g, p):
      pltpu.async_copy(
          bufs[p],
          out_hbm.at[pl.ds(out_base + g * group_rows, group_rows)],
          ssems[p],
      )

    def wait_store(g, p):
      pltpu.make_async_copy(
          bufs[p],
          out_hbm.at[pl.ds(out_base + g * group_rows, group_rows)],
          ssems[p],
      ).wait()

    fire_gather(0, 0)
    fire_gather(1, 1)

    @pl.loop(0, n_groups - 2, step=2)
    def _pair(g):
      drain_gather(g, 0)
      fire_store(g, 0)
      drain_gather(g + 1, 1)
      fire_store(g + 1, 1)
      wait_store(g, 0)
      fire_gather(g + 2, 0)
      wait_store(g + 1, 1)
      fire_gather(g + 3, 1)

    g_last = n_groups - 2
    drain_gather(g_last, 0)
    fire_store(g_last, 0)
    drain_gather(g_last + 1, 1)
    fire_store(g_last + 1, 1)
    wait_store(g_last, 0)
    wait_store(g_last + 1, 1)

  return gather_kernel


def kernel(input_ids, weight):
  bsz, seq = input_ids.shape
  vocab, dim = weight.shape
  n = bsz * seq
  assert n % (_NW * _G * _IROW) == 0
  idx2d = input_ids.reshape(n // _IROW, _IROW)
  out = _make_gather(n // _IROW, vocab, dim)(idx2d, weight)
  out3 = out.reshape(bsz, seq, dim)
  return with_layout_constraint(
      out3, Layout(major_to_minor=(0, 1, 2), tiling=())
  )
